# baseline (speedup 1.0000x reference)
import functools

import jax
import jax.numpy as jnp
from jax import lax
from jax.experimental import pallas as pl
from jax.experimental.pallas import tpu as pltpu
from jax.experimental.pallas import tpu_sc as plsc

B = 16384
D = 64
K = 20
VOC = 1000000
NC = 2
NS = 16
NW = NC * NS
NSLOT = B * (K + 1)
NTILE = 7813
TBASE = NTILE // NW
TEXTRA = NTILE - TBASE * NW

SR1 = 1024
NS1 = 31
CAP1 = 784
SCAP1 = 112

SR2 = 512
NB2 = 8
CAP2 = 12784
CAPB = 1776
SCAP2 = 448
CAPW = 22528

EROWS = 16400
SENT = 0x7FFFFFFF
VOC_AL = (VOC // 128) * 128

_mesh = plsc.VectorSubcoreMesh(
    core_axis_name="c", subcore_axis_name="s", num_cores=NC, num_subcores=NS)
_cparams = pltpu.CompilerParams(
    needs_layout_passes=False, use_tc_tiling_on_sc=True)


def _worker_range(wid):
    t0 = wid * TBASE + jnp.minimum(wid, TEXTRA)
    nt = TBASE + jnp.where(wid < TEXTRA, 1, 0)
    wlo = t0 * 128
    whi = jnp.minimum((t0 + nt) * 128, VOC)
    return wlo, whi


def _u_body(pos_u_hbm, ut_hbm, utail_hbm, emb_hbm,
            pu, staged, uh_v, uh_e, sh_c, sh_e, scat2d, rows_out, sem):
    wid = lax.axis_index("s") * NC + lax.axis_index("c")
    wlo, whi = _worker_range(wid)
    iota = lax.iota(jnp.int32, 16)

    pltpu.sync_copy(pos_u_hbm, pu)

    def scan_body(g, nh):
        v = pu[pl.ds(g * 16, 16)]
        m = jnp.logical_and(v >= wlo, v < whi)
        base = jnp.minimum(nh, CAP1 - 16)
        plsc.store_compressed(uh_v.at[pl.ds(base, 16)], v, mask=m)
        plsc.store_compressed(uh_e.at[pl.ds(base, 16)], g * 16 + iota, mask=m)
        return jnp.minimum(nh + plsc.all_reduce_population_count(m)[0], CAP1 - 16)

    nh = lax.fori_loop(0, B // 16, scan_body, jnp.int32(0))

    def sr_body(s, carry):
        lo = wlo + s * SR1
        hi = jnp.minimum(lo + SR1, whi)

        @pl.when(lo < whi)
        def _():
            is_tail = hi > VOC_AL
            c0 = pl.multiple_of(jnp.minimum(lo, VOC_AL - SR1), 128)
            sbase = jnp.where(is_tail, VOC - SR1, c0)

            @pl.when(jnp.logical_not(is_tail))
            def _():
                pltpu.async_copy(
                    ut_hbm.at[pl.ds(0, D), pl.ds(c0, SR1)], staged, sem).wait()

            @pl.when(is_tail)
            def _():
                pltpu.async_copy(utail_hbm, staged, sem).wait()

            def pre_body(t, c2):
                sh_e[pl.ds(t * 16, 16)] = jnp.full((16,), EROWS - 16, jnp.int32)
                return c2
            lax.fori_loop(0, (SCAP1 + 32) // 16, pre_body, 0)

            def sscan_body(g, ns):
                v = uh_v[pl.ds(g * 16, 16)]
                e = uh_e[pl.ds(g * 16, 16)]
                m = jnp.logical_and(v >= lo, v < hi)
                base = jnp.minimum(ns, SCAP1)
                plsc.store_compressed(sh_c.at[pl.ds(base, 16)], v - sbase,
                                      mask=m)
                plsc.store_compressed(sh_e.at[pl.ds(base, 16)], e, mask=m)
                return jnp.minimum(
                    ns + plsc.all_reduce_population_count(m)[0], SCAP1)

            ns = lax.fori_loop(0, (nh + 15) // 16, sscan_body, jnp.int32(0))
            for jj in range(8):
                scat2d[jj] = sh_e[pl.ds(jj * 16, 16)]

            def ext_body(h, c2):
                c = sh_c[pl.ds(h, 16)][0]
                cvec = jnp.full((16,), c, jnp.int32)
                for q in range(4):
                    rows_out[h, pl.ds(q * 16, 16)] = plsc.load_gather(
                        staged, [q * 16 + iota, cvec])
                return c2

            lax.fori_loop(0, ns, ext_body, 0)

            def scat_body(j, c2):
                pltpu.async_copy(
                    rows_out.at[pl.ds(j * 16, 16)],
                    emb_hbm.at[scat2d.at[j]], sem).wait()
                return c2

            lax.fori_loop(0, (ns + 15) // 16, scat_body, 0)
        return carry

    lax.fori_loop(0, NS1, sr_body, 0)


_sc_uphase = pl.kernel(
    _u_body,
    out_type=jax.ShapeDtypeStruct((EROWS, 2 * D), jnp.float32),
    mesh=_mesh,
    compiler_params=_cparams,
    scratch_types=[
        pltpu.VMEM((B,), jnp.int32),
        pltpu.VMEM((D, SR1), jnp.float32),
        pltpu.VMEM((CAP1 + 16,), jnp.int32),
        pltpu.VMEM((CAP1 + 16,), jnp.int32),
        pltpu.VMEM((SCAP1 + 32,), jnp.int32),
        pltpu.VMEM((SCAP1 + 32,), jnp.int32),
        pltpu.VMEM((8, 16), jnp.int32),
        pltpu.VMEM((SCAP1 + 16, 2 * D), jnp.float32),
        pltpu.SemaphoreType.DMA,
    ],
)


def _v_body(scanin_hbm, vt_hbm, vtail_hbm, emb_hbm, sc_pk_hbm, id_pk_hbm,
            scanbuf, wh, bk, nbbuf, sh_loc, sh_ue, staged, urows, ebuf,
            cumbuf, out_sc, out_id, sem, sem2):
    wid = lax.axis_index("s") * NC + lax.axis_index("c")
    wlo, whi = _worker_range(wid)
    iota = lax.iota(jnp.int32, 16)
    rowstart = iota * 16

    def pre_wh(g, c):
        wh[pl.ds(g * 16, 16)] = jnp.full((16,), SENT, jnp.int32)
        return c
    lax.fori_loop(0, (CAP2 + 16) // 16, pre_wh, 0)

    def pre_bk(g, c):
        bk[pl.ds(g * 16, 16)] = jnp.full((16,), SENT, jnp.int32)
        return c
    lax.fori_loop(0, NB2 * (CAPB + 16) // 16, pre_bk, 0)

    def chunk_body(cc, nh):
        pltpu.sync_copy(scanin_hbm.at[pl.ds(cc * 16384, 16384)], scanbuf)

        def scan_body(g, nh2):
            v = scanbuf[pl.ds(g * 16, 16)]
            u = scanbuf[pl.ds(8192 + g * 16, 16)]
            m = jnp.logical_and(v >= wlo, v < whi)
            h = jnp.bitwise_or(lax.shift_left(v - wlo, 15), u)
            base = jnp.minimum(nh2, CAP2 - 16)
            plsc.store_compressed(wh.at[pl.ds(base, 16)], h, mask=m)
            return jnp.minimum(
                nh2 + plsc.all_reduce_population_count(m)[0], CAP2 - 16)

        return lax.fori_loop(0, 512, scan_body, nh)

    nh = lax.fori_loop(0, NSLOT // 8192, chunk_body, jnp.int32(0))

    def place_body(g, counts):
        h = wh[pl.ds(g * 16, 16)]
        b = lax.shift_right_logical(h, 27)
        new = []
        for k in range(NB2):
            m = b == k
            base = jnp.minimum(counts[k], CAPB)
            plsc.store_compressed(
                bk.at[pl.ds(k * (CAPB + 16) + base, 16)], h, mask=m)
            new.append(jnp.minimum(
                counts[k] + plsc.all_reduce_population_count(m)[0], CAPB))
        return tuple(new)

    counts = lax.fori_loop(0, (nh + 15) // 16, place_body,
                           tuple(jnp.int32(0) for _ in range(NB2)))
    nbv = jnp.zeros((16,), jnp.int32)
    for k in range(NB2):
        nbv = jnp.where(iota == k, counts[k], nbv)
    nbbuf[pl.ds(0, 16)] = nbv

    def build_ebuf(j, slot):
        for t in range(8):
            uev = sh_ue[pl.ds(j * 128 + t * 16, 16)]
            ebuf[slot, pl.ds(t * 16, 16)] = jnp.where(
                uev < 0, EROWS - 16, lax.shift_right_logical(uev, 1))

    def flush(off, fil):
        do = fil == 2048
        obase = pl.multiple_of(wid * CAPW + off, 2048)

        @pl.when(do)
        def _():
            pltpu.sync_copy(out_sc, sc_pk_hbm.at[pl.ds(obase, 2048)])
            pltpu.sync_copy(out_id, id_pk_hbm.at[pl.ds(obase, 2048)])

        return (jnp.where(do, off + 2048, off), jnp.where(do, 0, fil))

    def bucket_body(k, carry):
        nbk = plsc.load_gather(nbbuf, [jnp.full((16,), k, jnp.int32)])[0]

        def sr_body(si, carry2):
            s = k * 8 + si
            lo = wlo + s * SR2
            hi = jnp.minimum(lo + SR2, whi)

            def do_block(carry3):
                off3, fil3 = carry3
                is_tail = hi > VOC_AL
                c0 = pl.multiple_of(jnp.minimum(lo, VOC_AL - SR2), 128)
                sbase = jnp.where(is_tail, VOC - SR2, c0)

                @pl.when(jnp.logical_not(is_tail))
                def _():
                    pltpu.async_copy(
                        vt_hbm.at[pl.ds(0, D), pl.ds(c0, SR2)],
                        staged, sem2).wait()

                @pl.when(is_tail)
                def _():
                    pltpu.async_copy(vtail_hbm, staged, sem2).wait()

                def pre_sh(t, c2):
                    sh_ue[pl.ds(t * 16, 16)] = jnp.full((16,), -1, jnp.int32)
                    return c2
                lax.fori_loop(0, (SCAP2 + 64) // 16, pre_sh, 0)

                lo_l = lo - wlo
                hi_l = hi - wlo
                dc = sbase - wlo

                def rescan_body(g, ns):
                    h = bk[pl.ds(k * (CAPB + 16) + g * 16, 16)]
                    loc = lax.shift_right_logical(h, 15)
                    m = jnp.logical_and(loc >= lo_l, loc < hi_l)
                    base = jnp.minimum(ns, SCAP2)
                    plsc.store_compressed(
                        sh_loc.at[pl.ds(base, 16)], loc - dc, mask=m)
                    plsc.store_compressed(
                        sh_ue.at[pl.ds(base, 16)],
                        jnp.bitwise_and(h, 32767), mask=m)
                    return jnp.minimum(
                        ns + plsc.all_reduce_population_count(m)[0], SCAP2)

                ns = lax.fori_loop(0, (nbk + 15) // 16, rescan_body,
                                   jnp.int32(0))

                build_ebuf(0, 0)
                cps = [None, None]
                cps[0] = pltpu.async_copy(
                    emb_hbm.at[ebuf.at[0]], urows.at[0], sem)
                off4, fil4 = off3, fil3
                for j in range(4):
                    live = j * 128 < ns

                    if j < 3:
                        def pref(jj=j):
                            build_ebuf(jj + 1, (jj + 1) & 1)
                        pl.when((j + 1) * 128 < ns)(pref)
                        cps[(j + 1) & 1] = pltpu.async_copy(
                            emb_hbm.at[ebuf.at[(j + 1) & 1]],
                            urows.at[(j + 1) & 1], sem)
                    cps[j & 1].wait()

                    def wave(off5, fil5, jj=j):
                        ur = urows.at[jj & 1]

                        def hit_body(h2, c2):
                            cl = sh_loc[pl.ds(jj * 128 + h2, 16)][0]
                            cvec = jnp.full((16,), cl, jnp.int32)
                            acc = plsc.load_gather(staged, [iota, cvec]) \
                                * ur[h2, pl.ds(0, 16)]
                            for q in range(1, 4):
                                acc = acc + plsc.load_gather(
                                    staged, [q * 16 + iota, cvec]) \
                                    * ur[h2, pl.ds(q * 16, 16)]
                            cumbuf[pl.ds(h2 * 16, 16)] = acc
                            return c2

                        lax.fori_loop(0, jnp.minimum(128, ns - jj * 128),
                                      hit_body, 0)

                        def fin_body(g, c2):
                            t2 = plsc.load_gather(cumbuf, [g * 256 + rowstart])
                            for t3 in range(1, 16):
                                t2 = t2 + plsc.load_gather(
                                    cumbuf, [g * 256 + rowstart + t3])
                            out_sc[pl.ds(fil5 + g * 16, 16)] = t2
                            return c2

                        lax.fori_loop(0, 8, fin_body, 0)
                        for t in range(8):
                            out_id[pl.ds(fil5 + t * 16, 16)] = \
                                sh_ue[pl.ds(jj * 128 + t * 16, 16)]
                        return off5, fil5 + 128

                    off4, fil4 = lax.cond(
                        live, wave, lambda o, f: (o, f), off4, fil4)
                    off4, fil4 = flush(off4, fil4)
                return off4, fil4

            return lax.cond(lo < whi, do_block, lambda c: c, carry2)

        return lax.fori_loop(0, 8, sr_body, carry)

    off, fil = lax.fori_loop(0, NB2, bucket_body,
                             (jnp.int32(0), jnp.int32(0)))

    def pad_body(t, c):
        out_id[pl.ds(fil + t * 16, 16)] = jnp.full((16,), -1, jnp.int32)
        return c

    lax.fori_loop(0, (2048 - fil) // 16, pad_body, 0)
    obase = pl.multiple_of(wid * CAPW + off, 2048)
    pltpu.sync_copy(out_sc, sc_pk_hbm.at[pl.ds(obase, 2048)])
    pltpu.sync_copy(out_id, id_pk_hbm.at[pl.ds(obase, 2048)])

    def pre_neg(g, c):
        out_id[pl.ds(g * 16, 16)] = jnp.full((16,), -1, jnp.int32)
        return c
    lax.fori_loop(0, 128, pre_neg, 0)

    def fill_body(t, c):
        fbase = pl.multiple_of(wid * CAPW + off + 2048 + t * 2048, 2048)
        pltpu.sync_copy(out_id, id_pk_hbm.at[pl.ds(fbase, 2048)])
        return c

    lax.fori_loop(0, (CAPW - off - 2048) // 2048, fill_body, 0)


_sc_vphase = pl.kernel(
    _v_body,
    out_type=(jax.ShapeDtypeStruct((NW * CAPW,), jnp.float32),
              jax.ShapeDtypeStruct((NW * CAPW,), jnp.int32)),
    mesh=_mesh,
    compiler_params=_cparams,
    scratch_types=[
        pltpu.VMEM((16384,), jnp.int32),
        pltpu.VMEM((CAP2 + 16,), jnp.int32),
        pltpu.VMEM((NB2 * (CAPB + 16),), jnp.int32),
        pltpu.VMEM((16,), jnp.int32),
        pltpu.VMEM((SCAP2 + 64,), jnp.int32),
        pltpu.VMEM((SCAP2 + 64,), jnp.int32),
        pltpu.VMEM((D, SR2), jnp.float32),
        pltpu.VMEM((2, 128, 2 * D), jnp.float32),
        pltpu.VMEM((2, 128), jnp.int32),
        pltpu.VMEM((2048,), jnp.float32),
        pltpu.VMEM((2048,), jnp.float32),
        pltpu.VMEM((2048,), jnp.int32),
        pltpu.SemaphoreType.DMA,
        pltpu.SemaphoreType.DMA,
    ],
)

_TC_ROWS = NW * CAPW // 128


def _tc_body(x_ref, id_ref, o_ref):
    x = x_ref[...]
    i = id_ref[...]
    valid = i >= 0
    is_pos = jnp.logical_and(valid, jnp.bitwise_and(i, 1) == 0)
    x = jnp.where(valid, x, 0.0)
    xc = jnp.clip(x, -10.0, 10.0)
    t = jnp.where(is_pos, -xc, xc)
    term = jnp.log1p(jnp.exp(t))
    term = jnp.where(valid, term, 0.0)
    pos_mean = jnp.sum(jnp.where(is_pos, term, 0.0)) * (1.0 / B)
    neg_mean = jnp.sum(jnp.where(is_pos, 0.0, term)) * (1.0 / (B * K))
    lane = lax.broadcasted_iota(jnp.int32, (1, 128), 1)
    o_ref[...] = jnp.where(lane == 0, pos_mean,
                           jnp.where(lane == 1, neg_mean, 0.0))


_tc_loss = pl.pallas_call(
    _tc_body,
    out_shape=jax.ShapeDtypeStruct((1, 128), jnp.float32),
)


def kernel(pos_u, pos_v, neg_v, u_table, v_table):
    ut = u_table.T
    vt = v_table.T
    vall = jnp.concatenate([pos_v, neg_v.reshape(-1)])
    elem = jnp.arange(B, dtype=jnp.int32)
    ue2 = jnp.concatenate(
        [elem * 2, (jnp.repeat(elem, K) * 2 + 1)]).astype(jnp.int32)
    scanin = jnp.concatenate(
        [vall.reshape(NSLOT // 8192, 8192), ue2.reshape(NSLOT // 8192, 8192)],
        axis=1).reshape(-1)
    utail = lax.slice(ut, (0, VOC - SR1), (D, VOC))
    vtail = lax.slice(vt, (0, VOC - SR2), (D, VOC))
    emb = _sc_uphase(pos_u, ut, utail)
    sc_pk, id_pk = _sc_vphase(scanin, vt, vtail, emb)
    sums = _tc_loss(sc_pk.reshape(_TC_ROWS, 128), id_pk.reshape(_TC_ROWS, 128))
    a = sums[0, 0]
    b = sums[0, 1]
    return (a + b, a, b)


# speedup vs baseline: 10.4702x; 10.4702x over previous
import functools

import jax
import jax.numpy as jnp
from jax import lax
from jax.experimental import pallas as pl
from jax.experimental.pallas import tpu as pltpu
from jax.experimental.pallas import tpu_sc as plsc

B = 16384
D = 64
K = 20
NC = 2
NS = 16
NW = NC * NS
EPW = B // NW
CH = 32
NCH = EPW // CH
DOTS = CH * (K + 1)
SCW = NCH * DOTS
TOT = NW * SCW
GQ = 6
GN = DOTS // GQ


def _sc_body(pos_u_hbm, v_idx_hbm, u_table_hbm, v_table_hbm, out_hbm,
             u_idx, v_idx, emb_u, rows, cumbuf, scores, sem):
    wid = lax.axis_index("s") * NC + lax.axis_index("c")
    pltpu.sync_copy(pos_u_hbm.at[pl.ds(wid * EPW, EPW)], u_idx)
    rowstart = lax.iota(jnp.int32, 16) * 16

    def issue(c, p):
        pltpu.sync_copy(v_idx_hbm.at[wid * NCH + c], v_idx.at[p])
        pltpu.async_copy(
            u_table_hbm.at[u_idx.at[pl.ds(c * CH, CH)]], emb_u.at[p],
            sem.at[p])
        for q in range(GQ):
            pltpu.async_copy(
                v_table_hbm.at[v_idx.at[p].at[pl.ds(q * GN, GN)]],
                rows.at[p].at[pl.ds(q * GN, GN)], sem.at[p])

    def await_chunk(c, p):
        pltpu.make_async_copy(
            u_table_hbm.at[u_idx.at[pl.ds(c * CH, CH)]], emb_u.at[p],
            sem.at[p]).wait()
        for q in range(GQ):
            pltpu.make_async_copy(
                v_table_hbm.at[v_idx.at[p].at[pl.ds(q * GN, GN)]],
                rows.at[p].at[pl.ds(q * GN, GN)], sem.at[p]).wait()

    issue(0, 0)

    def chunk_body(c, carry):
        p = c & 1

        @pl.when(c + 1 < NCH)
        def _():
            issue(c + 1, 1 - p)

        await_chunk(c, p)

        emb_c = emb_u.at[p]
        rows_c = rows.at[p]

        def pos_body(e, carry2):
            acc = emb_c[e, pl.ds(0, 16)] * rows_c[e, pl.ds(0, 16)]
            for q in range(1, 4):
                acc = acc + (emb_c[e, pl.ds(q * 16, 16)]
                             * rows_c[e, pl.ds(q * 16, 16)])
            cumbuf[pl.ds(e * 16, 16)] = acc
            return carry2

        lax.fori_loop(0, CH, pos_body, 0, unroll=4)

        def neg_body(e, carry2):
            u0 = emb_c[e, pl.ds(0, 16)]
            u1 = emb_c[e, pl.ds(16, 16)]
            u2 = emb_c[e, pl.ds(32, 16)]
            u3 = emb_c[e, pl.ds(48, 16)]
            rbase = CH + e * K
            for k in range(K):
                r = rbase + k
                acc = (u0 * rows_c[r, pl.ds(0, 16)]
                       + u1 * rows_c[r, pl.ds(16, 16)]
                       + u2 * rows_c[r, pl.ds(32, 16)]
                       + u3 * rows_c[r, pl.ds(48, 16)])
                cumbuf[pl.ds(r * 16, 16)] = acc
            return carry2

        lax.fori_loop(0, CH, neg_body, 0)

        sbase = c * DOTS

        def fin_body(g, carry2):
            base = g * 256
            t = plsc.load_gather(cumbuf, [base + rowstart])
            for j in range(1, 16):
                t = t + plsc.load_gather(cumbuf, [base + rowstart + j])
            scores[pl.ds(sbase + g * 16, 16)] = t
            return carry2

        lax.fori_loop(0, DOTS // 16, fin_body, 0, unroll=4)
        return carry

    lax.fori_loop(0, NCH, chunk_body, 0)
    pltpu.sync_copy(scores, out_hbm.at[pl.ds(wid * SCW, SCW)])


_sc_scores = pl.kernel(
    _sc_body,
    out_type=jax.ShapeDtypeStruct((TOT,), jnp.float32),
    mesh=plsc.VectorSubcoreMesh(
        core_axis_name="c", subcore_axis_name="s",
        num_cores=NC, num_subcores=NS),
    compiler_params=pltpu.CompilerParams(
        needs_layout_passes=False, use_tc_tiling_on_sc=False),
    scratch_types=[
        pltpu.VMEM((EPW,), jnp.int32),
        pltpu.VMEM((2, DOTS), jnp.int32),
        pltpu.VMEM((2, CH, D), jnp.float32),
        pltpu.VMEM((2, DOTS, D), jnp.float32),
        pltpu.VMEM((DOTS * 16,), jnp.float32),
        pltpu.VMEM((SCW,), jnp.float32),
        pltpu.SemaphoreType.DMA((2,)),
    ],
)

_TC_ROWS = TOT // 128


def _tc_body(x_ref, o_ref):
    x = x_ref[...]
    n = (lax.broadcasted_iota(jnp.int32, (_TC_ROWS, 128), 0) * 128
         + lax.broadcasted_iota(jnp.int32, (_TC_ROWS, 128), 1))
    r = n % DOTS
    is_pos = r < CH
    xc = jnp.clip(x, -10.0, 10.0)
    t = jnp.where(is_pos, -xc, xc)
    term = jnp.log1p(jnp.exp(t))
    pos_mean = jnp.sum(jnp.where(is_pos, term, 0.0)) * (1.0 / B)
    neg_mean = jnp.sum(jnp.where(is_pos, 0.0, term)) * (1.0 / (B * K))
    lane = lax.broadcasted_iota(jnp.int32, (1, 128), 1)
    o_ref[...] = jnp.where(lane == 0, pos_mean,
                           jnp.where(lane == 1, neg_mean, 0.0))


_tc_loss = pl.pallas_call(
    _tc_body,
    out_shape=jax.ShapeDtypeStruct((1, 128), jnp.float32),
)


def kernel(pos_u, pos_v, neg_v, u_table, v_table):
    v_idx = jnp.concatenate(
        [pos_v.reshape(B // CH, CH), neg_v.reshape(B // CH, CH * K)], axis=1)
    scores = _sc_scores(pos_u, v_idx, u_table, v_table)
    sums = _tc_loss(scores.reshape(_TC_ROWS, 128))
    a = sums[0, 0]
    b = sums[0, 1]
    return (a + b, a, b)


# speedup vs baseline: 10.4833x; 1.0013x over previous
import jax
import jax.numpy as jnp
from jax import lax
from jax.experimental import pallas as pl
from jax.experimental.pallas import tpu as pltpu
from jax.experimental.pallas import tpu_sc as plsc

B = 16384
D = 64
K = 20
NC = 2
NS = 16
NW = NC * NS
EPW = B // NW
CH = 32
NCH = EPW // CH
DOTS = CH * (K + 1)
SCW = NCH * DOTS
TOT = NW * SCW
GQ = 6
GN = DOTS // GQ


def _sc_body(pos_u_hbm, v_idx_hbm, u_table_hbm, v_table_hbm, out_hbm,
             u_idx, v_idx, emb_u, rows, cumbuf, scores, sem):
    wid = lax.axis_index("s") * NC + lax.axis_index("c")
    pltpu.sync_copy(pos_u_hbm.at[pl.ds(wid * EPW, EPW)], u_idx)
    rowstart = lax.iota(jnp.int32, 16) * 16

    def issue(c, p):
        pltpu.sync_copy(v_idx_hbm.at[wid * NCH + c], v_idx.at[p])
        pltpu.async_copy(
            u_table_hbm.at[u_idx.at[pl.ds(c * CH, CH)]], emb_u.at[p],
            sem.at[p])
        for q in range(GQ):
            pltpu.async_copy(
                v_table_hbm.at[v_idx.at[p].at[pl.ds(q * GN, GN)]],
                rows.at[p].at[pl.ds(q * GN, GN)], sem.at[p])

    def await_chunk(c, p):
        pltpu.make_async_copy(
            u_table_hbm.at[u_idx.at[pl.ds(c * CH, CH)]], emb_u.at[p],
            sem.at[p]).wait()
        for q in range(GQ):
            pltpu.make_async_copy(
                v_table_hbm.at[v_idx.at[p].at[pl.ds(q * GN, GN)]],
                rows.at[p].at[pl.ds(q * GN, GN)], sem.at[p]).wait()

    issue(0, 0)

    def chunk_body(c, carry):
        p = c & 1

        @pl.when(c + 1 < NCH)
        def _():
            issue(c + 1, 1 - p)

        await_chunk(c, p)

        emb_c = emb_u.at[p]
        rows_c = rows.at[p]

        def pos_body(e, carry2):
            acc = emb_c[e, pl.ds(0, 16)] * rows_c[e, pl.ds(0, 16)]
            for q in range(1, 4):
                acc = acc + (emb_c[e, pl.ds(q * 16, 16)]
                             * rows_c[e, pl.ds(q * 16, 16)])
            cumbuf[pl.ds(e * 16, 16)] = acc
            return carry2

        lax.fori_loop(0, CH, pos_body, 0, unroll=4)

        def neg_body(e, carry2):
            u0 = emb_c[e, pl.ds(0, 16)]
            u1 = emb_c[e, pl.ds(16, 16)]
            u2 = emb_c[e, pl.ds(32, 16)]
            u3 = emb_c[e, pl.ds(48, 16)]
            rbase = CH + e * K
            for k in range(K):
                r = rbase + k
                acc = (u0 * rows_c[r, pl.ds(0, 16)]
                       + u1 * rows_c[r, pl.ds(16, 16)]
                       + u2 * rows_c[r, pl.ds(32, 16)]
                       + u3 * rows_c[r, pl.ds(48, 16)])
                cumbuf[pl.ds(r * 16, 16)] = acc
            return carry2

        lax.fori_loop(0, CH, neg_body, 0, unroll=2)

        sbase = c * DOTS

        def fin_body(g, carry2):
            base = g * 256
            cols = [plsc.load_gather(cumbuf, [base + rowstart + j])
                    for j in range(16)]
            while len(cols) > 1:
                cols = [cols[i] + cols[i + 1] for i in range(0, len(cols), 2)]
            scores[pl.ds(sbase + g * 16, 16)] = cols[0]
            return carry2

        lax.fori_loop(0, DOTS // 16, fin_body, 0, unroll=4)
        return carry

    lax.fori_loop(0, NCH, chunk_body, 0)
    pltpu.sync_copy(scores, out_hbm.at[pl.ds(wid * SCW, SCW)])


_sc_scores = pl.kernel(
    _sc_body,
    out_type=jax.ShapeDtypeStruct((TOT,), jnp.float32),
    mesh=plsc.VectorSubcoreMesh(
        core_axis_name="c", subcore_axis_name="s",
        num_cores=NC, num_subcores=NS),
    compiler_params=pltpu.CompilerParams(
        needs_layout_passes=False, use_tc_tiling_on_sc=False),
    scratch_types=[
        pltpu.VMEM((EPW,), jnp.int32),
        pltpu.VMEM((2, DOTS), jnp.int32),
        pltpu.VMEM((2, CH, D), jnp.float32),
        pltpu.VMEM((2, DOTS, D), jnp.float32),
        pltpu.VMEM((DOTS * 16,), jnp.float32),
        pltpu.VMEM((SCW,), jnp.float32),
        pltpu.SemaphoreType.DMA((2,)),
    ],
)

_TC_ROWS = TOT // 128


def _tc_body(x_ref, o_ref):
    x = x_ref[...]
    n = (lax.broadcasted_iota(jnp.int32, (_TC_ROWS, 128), 0) * 128
         + lax.broadcasted_iota(jnp.int32, (_TC_ROWS, 128), 1))
    r = n % DOTS
    is_pos = r < CH
    xc = jnp.clip(x, -10.0, 10.0)
    t = jnp.where(is_pos, -xc, xc)
    term = jnp.log1p(jnp.exp(t))
    pos_mean = jnp.sum(jnp.where(is_pos, term, 0.0)) * (1.0 / B)
    neg_mean = jnp.sum(jnp.where(is_pos, 0.0, term)) * (1.0 / (B * K))
    lane = lax.broadcasted_iota(jnp.int32, (1, 128), 1)
    o_ref[...] = jnp.where(lane == 0, pos_mean,
                           jnp.where(lane == 1, neg_mean, 0.0))


_tc_loss = pl.pallas_call(
    _tc_body,
    out_shape=jax.ShapeDtypeStruct((1, 128), jnp.float32),
)


def kernel(pos_u, pos_v, neg_v, u_table, v_table):
    v_idx = jnp.concatenate(
        [pos_v.reshape(B // CH, CH), neg_v.reshape(B // CH, CH * K)], axis=1)
    scores = _sc_scores(pos_u, v_idx, u_table, v_table)
    sums = _tc_loss(scores.reshape(_TC_ROWS, 128))
    a = sums[0, 0]
    b = sums[0, 1]
    return (a + b, a, b)
